# Initial kernel scaffold; baseline (speedup 1.0000x reference)
import jax
import jax.numpy as jnp
from jax.experimental import pallas as pl


def kernel(x, conv_weight, conv_bias):
    raise NotImplementedError("write your pallas kernel here")


# speedup vs baseline: 6.5557x; 6.5557x over previous
import functools

import jax
import jax.numpy as jnp
from jax.experimental import pallas as pl
from jax.experimental.pallas import tpu as pltpu


def _fused_kernel(x_ref, d_ref, w_ref, b_ref, o_ref, *, width):
    x = x_ref[...].astype(jnp.bfloat16)
    c = x.shape[0]
    hw_up = o_ref.shape[-1]
    w_up = 2 * width

    xu = jnp.dot(x, d_ref[...],
                 preferred_element_type=jnp.float32).astype(jnp.bfloat16)

    col = jax.lax.broadcasted_iota(jnp.int32, (1, hw_up), 1) % w_up
    left_ok = (col >= 1).astype(jnp.bfloat16)
    right_ok = (col <= w_up - 2).astype(jnp.bfloat16)

    acc = jnp.zeros((c, hw_up), jnp.float32)
    for ky in range(3):
        for kx in range(3):
            s = (ky - 1) * w_up + (kx - 1)
            if s > 0:
                xs = jnp.concatenate(
                    [xu[:, s:], jnp.zeros((c, s), xu.dtype)], axis=1)
            elif s < 0:
                xs = jnp.concatenate(
                    [jnp.zeros((c, -s), xu.dtype), xu[:, :hw_up + s]], axis=1)
            else:
                xs = xu
            if kx == 0:
                xs = xs * left_ok
            elif kx == 2:
                xs = xs * right_ok
            acc = acc + jnp.dot(w_ref[ky, kx], xs,
                                preferred_element_type=jnp.float32)
    o_ref[...] = (acc + b_ref[...]).astype(o_ref.dtype)


def kernel(x, conv_weight, conv_bias):
    n, c, h, w = x.shape
    hw = h * w

    k_i = jnp.arange(hw)[:, None]
    m_i = jnp.arange(4 * hw)[None, :]
    src = (m_i // (2 * w)) // 2 * w + (m_i % (2 * w)) // 2
    d = (k_i == src).astype(jnp.bfloat16)

    wk = jnp.transpose(conv_weight, (2, 3, 0, 1)).astype(jnp.bfloat16)
    b2 = conv_bias.reshape(c, 1)
    x2 = x.reshape(n, c, hw)

    out = pl.pallas_call(
        functools.partial(_fused_kernel, width=w),
        out_shape=jax.ShapeDtypeStruct((n, c, 4 * hw), jnp.float32),
        grid=(n,),
        in_specs=[
            pl.BlockSpec((None, c, hw), lambda i: (i, 0, 0)),
            pl.BlockSpec((hw, 4 * hw), lambda i: (0, 0)),
            pl.BlockSpec((3, 3, c, c), lambda i: (0, 0, 0, 0)),
            pl.BlockSpec((c, 1), lambda i: (0, 0)),
        ],
        out_specs=pl.BlockSpec((None, c, 4 * hw), lambda i: (i, 0, 0)),
        compiler_params=pltpu.CompilerParams(
            dimension_semantics=("parallel",),
            vmem_limit_bytes=64 * 1024 * 1024,
        ),
    )(x2, d, wk, b2)
    return out.reshape(n, c, 2 * h, 2 * w)


# speedup vs baseline: 6.6645x; 1.0166x over previous
import functools

import jax
import jax.numpy as jnp
import numpy as np
from jax.experimental import pallas as pl
from jax.experimental.pallas import tpu as pltpu


def _fused_kernel(x_ref, d_ref, w_ref, b_ref, o_ref, *, width):
    x = x_ref[...].astype(jnp.bfloat16)
    c = x.shape[0]
    hw_up = o_ref.shape[-1]
    w_up = 2 * width

    xu = jnp.dot(x, d_ref[...],
                 preferred_element_type=jnp.float32).astype(jnp.bfloat16)

    col = jax.lax.broadcasted_iota(jnp.int32, (1, hw_up), 1) % w_up
    left_ok = (col >= 1).astype(jnp.bfloat16)
    right_ok = (col <= w_up - 2).astype(jnp.bfloat16)

    acc = jnp.zeros((c, hw_up), jnp.float32)
    for ky in range(3):
        for kx in range(3):
            s = (ky - 1) * w_up + (kx - 1)
            if s > 0:
                xs = jnp.concatenate(
                    [xu[:, s:], jnp.zeros((c, s), xu.dtype)], axis=1)
            elif s < 0:
                xs = jnp.concatenate(
                    [jnp.zeros((c, -s), xu.dtype), xu[:, :hw_up + s]], axis=1)
            else:
                xs = xu
            if kx == 0:
                xs = xs * left_ok
            elif kx == 2:
                xs = xs * right_ok
            acc = acc + jnp.dot(w_ref[ky, kx], xs,
                                preferred_element_type=jnp.float32)
    o_ref[...] = (acc + b_ref[...]).astype(o_ref.dtype)


def kernel(x, conv_weight, conv_bias):
    n, c, h, w = x.shape
    hw = h * w

    k_i = np.arange(hw)[:, None]
    m_i = np.arange(4 * hw)[None, :]
    src = (m_i // (2 * w)) // 2 * w + (m_i % (2 * w)) // 2
    d = jnp.asarray(k_i == src, dtype=jnp.bfloat16)

    wk = jnp.transpose(conv_weight, (2, 3, 0, 1)).astype(jnp.bfloat16)
    b2 = conv_bias.reshape(c, 1)
    x2 = x.reshape(n, c, hw)

    out = pl.pallas_call(
        functools.partial(_fused_kernel, width=w),
        out_shape=jax.ShapeDtypeStruct((n, c, 4 * hw), jnp.float32),
        grid=(n,),
        in_specs=[
            pl.BlockSpec((None, c, hw), lambda i: (i, 0, 0)),
            pl.BlockSpec((hw, 4 * hw), lambda i: (0, 0)),
            pl.BlockSpec((3, 3, c, c), lambda i: (0, 0, 0, 0)),
            pl.BlockSpec((c, 1), lambda i: (0, 0)),
        ],
        out_specs=pl.BlockSpec((None, c, 4 * hw), lambda i: (i, 0, 0)),
        compiler_params=pltpu.CompilerParams(
            dimension_semantics=("parallel",),
            vmem_limit_bytes=64 * 1024 * 1024,
        ),
    )(x2, d, wk, b2)
    return out.reshape(n, c, 2 * h, 2 * w)
